# baseline (speedup 1.0000x reference)
import jax
import jax.numpy as jnp
from jax.experimental import pallas as pl

N, E, F, L, G, B = 10000, 320000, 128, 4, 7, 64


def _mm_relu_body(h_ref, w_ref, b_ref, out_ref):
    out_ref[...] = jnp.maximum(
        jnp.dot(h_ref[...], w_ref[...],
                preferred_element_type=jnp.float32) + b_ref[...], 0.0)


_mm_relu = pl.pallas_call(
    _mm_relu_body, out_shape=jax.ShapeDtypeStruct((N, F), jnp.float32))


def _mm_body(h_ref, w_ref, b_ref, out_ref):
    out_ref[...] = jnp.dot(h_ref[...], w_ref[...],
                           preferred_element_type=jnp.float32) + b_ref[...]


_mm = pl.pallas_call(
    _mm_body, out_shape=jax.ShapeDtypeStruct((N, F), jnp.float32))


def _head_body(r_ref, wp1_ref, bp1_ref, wp2_ref, bp2_ref, out_ref):
    f32 = jnp.float32
    a = jnp.maximum(
        jnp.dot(r_ref[...], wp1_ref[...], preferred_element_type=f32)
        + bp1_ref[...], 0.0)
    out_ref[...] = (jnp.dot(a, wp2_ref[...], preferred_element_type=f32)
                    + bp2_ref[...])


_head = pl.pallas_call(
    _head_body, out_shape=jax.ShapeDtypeStruct((G * B, F), jnp.float32))


def kernel(x, edge_index, batch, W1, b1, W2, b2, gn_scale, gn_bias,
           Wp1, bp1, Wp2, bp2):
    f32 = jnp.float32
    x = x.astype(f32)
    src = edge_index[0]
    dst = edge_index[1]
    counts = jax.ops.segment_sum(jnp.ones((N,), f32), batch, num_segments=B)
    denom = jnp.maximum(counts - 1.0, 1.0)

    def gin(h, g, l):
        agg = jax.ops.segment_sum(h[src], dst, num_segments=N)
        h2 = h + agg
        h2 = _mm_relu(h2, W1[g, l].astype(f32), b1[g, l].astype(f32))
        h2 = _mm(h2, W2[g, l].astype(f32), b2[g, l].astype(f32))
        mean = jax.ops.segment_sum(h2, batch, num_segments=B) \
            / jnp.maximum(counts, 1.0)[:, None]
        diff = h2 - mean[batch]
        var = jax.ops.segment_sum(diff * diff, batch, num_segments=B) \
            / denom[:, None]
        std = jnp.sqrt(var)
        h2 = (h2 - mean[batch]) / (std[batch] + 1e-05)
        return gn_scale[g, l] * h2 + gn_bias[g, l]

    pools = []
    for g in range(G):
        h = x
        for l in range(L):
            h = gin(h, g, l)
            pools.append(jax.ops.segment_sum(h, batch, num_segments=B))

    r = jnp.stack(pools).reshape(G, L, B, F)
    r = r.transpose(0, 2, 1, 3).reshape(G * B, L * F)
    out = _head(r, Wp1.astype(f32), bp1.astype(f32),
                Wp2.astype(f32), bp2.astype(f32))
    out = out.reshape(G, B, F)
    return out[0], out[1:]
